# baseline (speedup 1.0000x reference)
import jax
import jax.numpy as jnp
from jax import lax
from jax.experimental import pallas as pl
from jax.experimental.pallas import tpu as pltpu
from jax.experimental.pallas import tpu_sc as plsc

B = 4096
V = 100000
NS = 16
L = 16
SPW = 2
SW = 128
BPW = SPW * SW
NV = SW // L


def _sc_body(inpt_hbm, tgt_hbm, out_hbm, tgt_a, tgt_b, seg_v, acc_v, sem):
    sid = lax.axis_index("s")
    lane_iota = lax.iota(jnp.int32, L)
    acc = jnp.zeros((L,), jnp.float32)
    for s, tgt_v in ((0, tgt_a), (1, tgt_b)):
        base = pl.multiple_of((sid * SPW + s) * SW, SW)
        pltpu.sync_copy(tgt_hbm.at[pl.ds(base, SW)], tgt_v)
        pltpu.async_copy(
            inpt_hbm.at[tgt_v, pl.ds(base, SW)], seg_v.at[s], sem
        ).wait()
        for i in range(NV):
            t16 = tgt_v[pl.ds(i * L, L)]
            diag = lane_iota + i * L
            vals = plsc.load_gather(seg_v, [jnp.full((L,), s, jnp.int32),
                                            diag, diag])
            acc = acc + jnp.where(t16 > 0, vals, jnp.float32(0.0))
    acc_v[...] = acc
    pltpu.sync_copy(acc_v, out_hbm.at[sid])


@jax.jit
def kernel(input, target):
    tgt = target.reshape(B).astype(jnp.int32)
    mesh = plsc.VectorSubcoreMesh(
        core_axis_name="c", subcore_axis_name="s", num_cores=1
    )
    parts = pl.kernel(
        _sc_body,
        out_type=jax.ShapeDtypeStruct((NS, L), jnp.float32),
        mesh=mesh,
        compiler_params=pltpu.CompilerParams(needs_layout_passes=False),
        scratch_types=[
            pltpu.VMEM((SW,), jnp.int32),
            pltpu.VMEM((SW,), jnp.int32),
            pltpu.VMEM((SPW, SW, SW), jnp.float32),
            pltpu.VMEM((L,), jnp.float32),
            pltpu.SemaphoreType.DMA,
        ],
    )(input.T, tgt)
    return -jnp.sum(parts)


# speedup vs baseline: 1.0672x; 1.0672x over previous
import jax
import jax.numpy as jnp
from jax import lax
from jax.experimental import pallas as pl
from jax.experimental.pallas import tpu as pltpu
from jax.experimental.pallas import tpu_sc as plsc

B = 4096
V = 100000
NC = 2
NS = 16
L = 16
NW = NC * NS
BPW = B // NW
NV = BPW // L
H = BPW // 2


def _sc_body(inpt_hbm, tgt_hbm, out_hbm, tgt_a, tgt_b, seg_v, acc_v,
             sem_a, sem_b):
    wid = lax.axis_index("s") * NC + lax.axis_index("c")
    base = pl.multiple_of(wid * BPW, BPW)
    pltpu.sync_copy(tgt_hbm.at[pl.ds(base, H)], tgt_a)
    pltpu.sync_copy(tgt_hbm.at[pl.ds(base + H, H)], tgt_b)
    copy_a = pltpu.async_copy(
        inpt_hbm.at[tgt_a, pl.ds(base, BPW)], seg_v.at[pl.ds(0, H)], sem_a
    )
    copy_b = pltpu.async_copy(
        inpt_hbm.at[tgt_b, pl.ds(base, BPW)], seg_v.at[pl.ds(H, H)], sem_b
    )
    lane_iota = lax.iota(jnp.int32, L)
    acc = jnp.zeros((L,), jnp.float32)
    copy_a.wait()
    for i in range(NV // 2):
        t16 = tgt_a[pl.ds(i * L, L)]
        diag = lane_iota + i * L
        vals = plsc.load_gather(seg_v, [diag, diag])
        acc = acc - jnp.where(t16 > 0, vals, jnp.float32(0.0))
    copy_b.wait()
    for i in range(NV // 2):
        t16 = tgt_b[pl.ds(i * L, L)]
        diag = lane_iota + H + i * L
        vals = plsc.load_gather(seg_v, [diag, diag])
        acc = acc - jnp.where(t16 > 0, vals, jnp.float32(0.0))
    acc_v[...] = acc
    pltpu.sync_copy(acc_v, out_hbm.at[wid])


@jax.jit
def kernel(input, target):
    tgt = target.reshape(B).astype(jnp.int32)
    mesh = plsc.VectorSubcoreMesh(core_axis_name="c", subcore_axis_name="s")
    parts = pl.kernel(
        _sc_body,
        out_type=jax.ShapeDtypeStruct((NW, L), jnp.float32),
        mesh=mesh,
        compiler_params=pltpu.CompilerParams(needs_layout_passes=False),
        scratch_types=[
            pltpu.VMEM((H,), jnp.int32),
            pltpu.VMEM((H,), jnp.int32),
            pltpu.VMEM((BPW, BPW), jnp.float32),
            pltpu.VMEM((L,), jnp.float32),
            pltpu.SemaphoreType.DMA,
            pltpu.SemaphoreType.DMA,
        ],
    )(input.T, tgt)
    return jnp.sum(parts)


# speedup vs baseline: 1.0971x; 1.0281x over previous
import jax
import jax.numpy as jnp
from jax import lax
from jax.experimental import pallas as pl
from jax.experimental.pallas import tpu as pltpu
from jax.experimental.pallas import tpu_sc as plsc

B = 4096
V = 100000
NC = 2
NS = 16
L = 16
NW = NC * NS
BPW = B // NW
NV = BPW // L
HV = NV // 2


def _sc_body(inpt_hbm, tgt_hbm, out_hbm, tgt_v, seg_v, acc_v, sem_a, sem_b):
    wid = lax.axis_index("s") * NC + lax.axis_index("c")
    base = pl.multiple_of(wid * BPW, BPW)
    pltpu.sync_copy(tgt_hbm.at[pl.ds(base, BPW)], tgt_v)
    t16s = [tgt_v[pl.ds(i * L, L)] for i in range(NV)]
    copies = [
        pltpu.async_copy(
            inpt_hbm.at[t16s[i], pl.ds(base, BPW)],
            seg_v.at[pl.ds(i * L, L)],
            sem_a if i < HV else sem_b,
        )
        for i in range(NV)
    ]
    lane_iota = lax.iota(jnp.int32, L)
    acc = jnp.zeros((L,), jnp.float32)
    for h in range(2):
        for c in copies[h * HV : (h + 1) * HV]:
            c.wait()
        for i in range(h * HV, (h + 1) * HV):
            diag = lane_iota + i * L
            vals = plsc.load_gather(seg_v, [diag, diag])
            acc = acc - jnp.where(t16s[i] > 0, vals, jnp.float32(0.0))
    acc_v[...] = acc
    pltpu.sync_copy(acc_v, out_hbm.at[wid])


@jax.jit
def kernel(input, target):
    tgt = target.reshape(B).astype(jnp.int32)
    mesh = plsc.VectorSubcoreMesh(core_axis_name="c", subcore_axis_name="s")
    parts = pl.kernel(
        _sc_body,
        out_type=jax.ShapeDtypeStruct((NW, L), jnp.float32),
        mesh=mesh,
        compiler_params=pltpu.CompilerParams(needs_layout_passes=False),
        scratch_types=[
            pltpu.VMEM((BPW,), jnp.int32),
            pltpu.VMEM((BPW, BPW), jnp.float32),
            pltpu.VMEM((L,), jnp.float32),
            pltpu.SemaphoreType.DMA,
            pltpu.SemaphoreType.DMA,
        ],
    )(input.T, tgt)
    return jnp.sum(parts)
